# Initial kernel scaffold; baseline (speedup 1.0000x reference)
import jax
import jax.numpy as jnp
from jax.experimental import pallas as pl


def kernel(values, cu_seqlens, feature_idx, token_table, embed_table, W_out):
    raise NotImplementedError("write your pallas kernel here")


# speedup vs baseline: 41.5480x; 41.5480x over previous
import functools

import jax
import jax.numpy as jnp
from jax import lax
from jax.experimental import pallas as pl
from jax.experimental.pallas import tpu as pltpu
from jax.experimental.pallas import tpu_sc as plsc

_T = 16384
_BP = 64
_NSEG = 16
_NC = 2
_NS = 16
_NW = _NC * _NS
_CHUNK = _T // _NW
_NIDX = _CHUNK // 128


def _sc_scores_kernel():
    mesh = plsc.VectorSubcoreMesh(
        core_axis_name="c", subcore_axis_name="s",
        num_cores=_NC, num_subcores=_NS)

    @functools.partial(
        pl.kernel,
        mesh=mesh,
        out_type=jax.ShapeDtypeStruct((_T,), jnp.float32),
        scratch_types=[
            pltpu.VMEM((_NIDX, 128), jnp.int32),
            pltpu.VMEM((_CHUNK, _BP), jnp.int32),
            pltpu.VMEM((256,), jnp.float32),
            pltpu.VMEM((128,), jnp.float32),
            pltpu.VMEM((128,), jnp.float32),
            pltpu.VMEM((128,), jnp.float32),
            pltpu.VMEM((_CHUNK,), jnp.float32),
            pltpu.SemaphoreType.DMA,
        ],
        compiler_params=pltpu.CompilerParams(
            needs_layout_passes=False, use_tc_tiling_on_sc=False),
    )
    def body(feat_hbm, table_hbm, emb_hbm, wout_hbm, s_hbm,
             idx_v, rows_v, emb_v, wout_v, ehi_v, elo_v, s_v, sem):
        wid = lax.axis_index("s") * _NC + lax.axis_index("c")
        base = wid * _CHUNK

        for j in range(_NIDX):
            pltpu.sync_copy(feat_hbm.at[pl.ds(base + j * 128, 128)],
                            idx_v.at[j])
        copies = [
            pltpu.async_copy(table_hbm.at[idx_v.at[j]],
                             rows_v.at[pl.ds(j * 128, 128)], sem)
            for j in range(_NIDX)
        ]

        pltpu.sync_copy(emb_hbm, emb_v.at[pl.ds(0, 192)])
        pltpu.sync_copy(wout_hbm, wout_v.at[pl.ds(0, 32)])
        io = lax.broadcasted_iota(jnp.int32, (16,), 0)
        w0 = wout_v[pl.ds(0, 16)]
        w1 = wout_v[pl.ds(16, 16)]
        e_vec = jnp.zeros((16,), jnp.float32)
        for v in range(6):
            r0 = emb_v[pl.ds(v * 32, 16)]
            r1 = emb_v[pl.ds(v * 32 + 16, 16)]
            dot_v = jnp.sum(r0 * w0 + r1 * w1)
            e_vec = e_vec + jnp.where(io == v, dot_v, 0.0)
        e_vec = e_vec * (1.0 / _BP)
        bits = plsc.bitcast(e_vec, jnp.int32)
        rounded = (bits + 0x7FFF + ((bits >> 16) & 1)) & ~0xFFFF
        e_hi = plsc.bitcast(rounded, jnp.float32)
        ehi_v[pl.ds(0, 16)] = e_hi
        elo_v[pl.ds(0, 16)] = e_vec - e_hi

        for c in copies:
            c.wait()

        def grp(g, carry):
            rows = g * 16 + io
            acc = jnp.zeros((16,), jnp.float32)
            for bp in range(_BP):
                col = jnp.full((16,), bp, jnp.int32)
                toks = plsc.load_gather(rows_v, [rows, col])
                acc = acc + plsc.load_gather(ehi_v, [toks])
                acc = acc + plsc.load_gather(elo_v, [toks])
            s_v[pl.ds(g * 16, 16)] = acc
            return carry

        lax.fori_loop(0, _CHUNK // 16, grp, 0)
        pltpu.sync_copy(s_v, s_hbm.at[pl.ds(base, _CHUNK)])

    return body


def _tc_combine(cu_ref, vals_ref, s_ref, out_ref):
    logv = jnp.log(vals_ref[...] + 1e-6)
    sv = s_ref[...]
    ls = logv * sv
    r_io = lax.broadcasted_iota(jnp.int32, (128, 128), 0)
    c_io = lax.broadcasted_iota(jnp.int32, (128, 128), 1)
    t_idx = r_io * 128 + c_io
    row16 = lax.broadcasted_iota(jnp.int32, (_NSEG, 128), 0)
    col16 = lax.broadcasted_iota(jnp.int32, (_NSEG, 128), 1)
    acc = jnp.zeros((_NSEG, 128), jnp.float32)
    for b in range(_NSEG):
        lo = cu_ref[b]
        hi = cu_ref[b + 1]
        m = ((t_idx >= lo) & (t_idx < hi)).astype(jnp.float32)
        cnt = jnp.sum(m)
        lsum = jnp.sum(m * logv)
        ssum = jnp.sum(m * sv)
        asum = jnp.sum(m * ls)
        res = asum - (lsum / jnp.maximum(cnt, 1.0)) * ssum
        acc = acc + jnp.where((row16 == b) & (col16 == 0), res, 0.0)
    out_ref[...] = acc


def kernel(values, cu_seqlens, feature_idx, token_table, embed_table, W_out):
    s = _sc_scores_kernel()(
        feature_idx, token_table,
        embed_table.reshape(-1), W_out.reshape(-1))
    out = pl.pallas_call(
        _tc_combine,
        out_shape=jax.ShapeDtypeStruct((_NSEG, 128), jnp.float32),
        in_specs=[
            pl.BlockSpec(memory_space=pltpu.SMEM),
            pl.BlockSpec(memory_space=pltpu.VMEM),
            pl.BlockSpec(memory_space=pltpu.VMEM),
        ],
    )(cu_seqlens, values.reshape(128, 128), s.reshape(128, 128))
    return out[:, :1]


# speedup vs baseline: 45.5399x; 1.0961x over previous
import functools

import jax
import jax.numpy as jnp
from jax import lax
from jax.experimental import pallas as pl
from jax.experimental.pallas import tpu as pltpu
from jax.experimental.pallas import tpu_sc as plsc

_T = 16384
_BP = 64
_NSEG = 16
_NC = 2
_NS = 16
_NW = _NC * _NS
_CHUNK = _T // _NW
_NIDX = _CHUNK // 128


def _sc_scores_kernel():
    mesh = plsc.VectorSubcoreMesh(
        core_axis_name="c", subcore_axis_name="s",
        num_cores=_NC, num_subcores=_NS)

    @functools.partial(
        pl.kernel,
        mesh=mesh,
        out_type=jax.ShapeDtypeStruct((_T,), jnp.float32),
        scratch_types=[
            pltpu.VMEM((_NIDX, 128), jnp.int32),
            pltpu.VMEM((_NIDX, 128), jnp.int32),
            pltpu.VMEM((_CHUNK,), jnp.int32),
            pltpu.VMEM((_CHUNK, 128), jnp.int32),
            pltpu.VMEM((256,), jnp.float32),
            pltpu.VMEM((128,), jnp.float32),
            pltpu.VMEM((128,), jnp.float32),
            pltpu.VMEM((128,), jnp.float32),
            pltpu.VMEM((_CHUNK,), jnp.float32),
            pltpu.SemaphoreType.DMA,
        ],
        compiler_params=pltpu.CompilerParams(needs_layout_passes=False),
    )
    def body(feat_hbm, table_hbm, emb_hbm, wout_hbm, s_hbm,
             idx_v, ridx_v, par_v, rows_v, emb_v, wout_v, ehi_v, elo_v,
             s_v, sem):
        wid = lax.axis_index("s") * _NC + lax.axis_index("c")
        base = wid * _CHUNK

        for j in range(_NIDX):
            pltpu.sync_copy(feat_hbm.at[pl.ds(base + j * 128, 128)],
                            idx_v.at[j])
        for j in range(_NIDX):
            for k in range(8):
                v = idx_v[j, pl.ds(k * 16, 16)]
                ridx_v[j, pl.ds(k * 16, 16)] = v >> 1
                par_v[pl.ds(j * 128 + k * 16, 16)] = (v & 1) * _BP

        copies = [
            pltpu.async_copy(table_hbm.at[ridx_v.at[j]],
                             rows_v.at[pl.ds(j * 128, 128)], sem)
            for j in range(_NIDX)
        ]

        pltpu.sync_copy(emb_hbm, emb_v.at[pl.ds(0, 192)])
        pltpu.sync_copy(wout_hbm, wout_v.at[pl.ds(0, 32)])
        io = lax.broadcasted_iota(jnp.int32, (16,), 0)
        w0 = wout_v[pl.ds(0, 16)]
        w1 = wout_v[pl.ds(16, 16)]
        e_vec = jnp.zeros((16,), jnp.float32)
        for v in range(6):
            r0 = emb_v[pl.ds(v * 32, 16)]
            r1 = emb_v[pl.ds(v * 32 + 16, 16)]
            dot_v = jnp.sum(r0 * w0 + r1 * w1)
            e_vec = e_vec + jnp.where(io == v, dot_v, 0.0)
        e_vec = e_vec * (1.0 / _BP)
        bits = plsc.bitcast(e_vec, jnp.int32)
        rounded = (bits + 0x7FFF + ((bits >> 16) & 1)) & ~0xFFFF
        e_hi = plsc.bitcast(rounded, jnp.float32)
        ehi_v[pl.ds(0, 16)] = e_hi
        elo_v[pl.ds(0, 16)] = e_vec - e_hi

        for j in range(_NIDX):
            copies[j].wait()

            def grp(g, carry, j=j):
                r0 = j * 128 + g * 16
                rows = r0 + io
                par = par_v[pl.ds(r0, 16)]
                acc = jnp.zeros((16,), jnp.float32)
                for bp in range(_BP):
                    col = par + ((io + bp) & (_BP - 1))
                    toks = plsc.load_gather(rows_v, [rows, col])
                    acc = acc + plsc.load_gather(ehi_v, [toks])
                    acc = acc + plsc.load_gather(elo_v, [toks])
                s_v[pl.ds(r0, 16)] = acc
                return carry

            lax.fori_loop(0, 128 // 16, grp, 0)

        pltpu.sync_copy(s_v, s_hbm.at[pl.ds(base, _CHUNK)])

    return body


def _tc_combine(cu_ref, vals_ref, s_ref, out_ref):
    logv = jnp.log(vals_ref[...] + 1e-6)
    sv = s_ref[...]
    ls = logv * sv
    r_io = lax.broadcasted_iota(jnp.int32, (128, 128), 0)
    c_io = lax.broadcasted_iota(jnp.int32, (128, 128), 1)
    t_idx = r_io * 128 + c_io
    row16 = lax.broadcasted_iota(jnp.int32, (_NSEG, 128), 0)
    col16 = lax.broadcasted_iota(jnp.int32, (_NSEG, 128), 1)
    acc = jnp.zeros((_NSEG, 128), jnp.float32)
    for b in range(_NSEG):
        lo = cu_ref[b]
        hi = cu_ref[b + 1]
        m = ((t_idx >= lo) & (t_idx < hi)).astype(jnp.float32)
        cnt = jnp.sum(m)
        lsum = jnp.sum(m * logv)
        ssum = jnp.sum(m * sv)
        asum = jnp.sum(m * ls)
        res = asum - (lsum / jnp.maximum(cnt, 1.0)) * ssum
        acc = acc + jnp.where((row16 == b) & (col16 == 0), res, 0.0)
    out_ref[...] = acc


def kernel(values, cu_seqlens, feature_idx, token_table, embed_table, W_out):
    table2 = token_table.reshape(token_table.shape[0] // 2, 2 * _BP)
    s = _sc_scores_kernel()(
        feature_idx, table2,
        embed_table.reshape(-1), W_out.reshape(-1))
    out = pl.pallas_call(
        _tc_combine,
        out_shape=jax.ShapeDtypeStruct((_NSEG, 128), jnp.float32),
        in_specs=[
            pl.BlockSpec(memory_space=pltpu.SMEM),
            pl.BlockSpec(memory_space=pltpu.VMEM),
            pl.BlockSpec(memory_space=pltpu.VMEM),
        ],
    )(cu_seqlens, values.reshape(128, 128), s.reshape(128, 128))
    return out[:, :1]


# speedup vs baseline: 46.6901x; 1.0253x over previous
import functools

import jax
import jax.numpy as jnp
from jax import lax
from jax.experimental import pallas as pl
from jax.experimental.pallas import tpu as pltpu
from jax.experimental.pallas import tpu_sc as plsc

_T = 16384
_BP = 64
_NSEG = 16
_NC = 2
_NS = 16
_NW = _NC * _NS
_CHUNK = _T // _NW
_NIDX = _CHUNK // 128


def _sc_scores_kernel():
    mesh = plsc.VectorSubcoreMesh(
        core_axis_name="c", subcore_axis_name="s",
        num_cores=_NC, num_subcores=_NS)

    @functools.partial(
        pl.kernel,
        mesh=mesh,
        out_type=jax.ShapeDtypeStruct((_T,), jnp.float32),
        scratch_types=[
            pltpu.VMEM((_NIDX, 128), jnp.int32),
            pltpu.VMEM((_NIDX, 128), jnp.int32),
            pltpu.VMEM((_CHUNK,), jnp.int32),
            pltpu.VMEM((_CHUNK, 128), jnp.int32),
            pltpu.VMEM((256,), jnp.float32),
            pltpu.VMEM((128,), jnp.float32),
            pltpu.VMEM((128,), jnp.int32),
            pltpu.VMEM((_CHUNK,), jnp.float32),
            pltpu.SemaphoreType.DMA,
        ],
        compiler_params=pltpu.CompilerParams(needs_layout_passes=False),
    )
    def body(feat_hbm, table_hbm, emb_hbm, wout_hbm, s_hbm,
             idx_v, ridx_v, par_v, rows_v, emb_v, wout_v, elut_v, s_v, sem):
        wid = lax.axis_index("s") * _NC + lax.axis_index("c")
        base = wid * _CHUNK

        for j in range(_NIDX):
            pltpu.sync_copy(feat_hbm.at[pl.ds(base + j * 128, 128)],
                            idx_v.at[j])
        for j in range(_NIDX):
            for k in range(8):
                v = idx_v[j, pl.ds(k * 16, 16)]
                ridx_v[j, pl.ds(k * 16, 16)] = v >> 1
                par_v[pl.ds(j * 128 + k * 16, 16)] = (v & 1) * _BP

        copies = [
            pltpu.async_copy(table_hbm.at[ridx_v.at[j]],
                             rows_v.at[pl.ds(j * 128, 128)], sem)
            for j in range(_NIDX)
        ]

        pltpu.sync_copy(emb_hbm, emb_v.at[pl.ds(0, 192)])
        pltpu.sync_copy(wout_hbm, wout_v.at[pl.ds(0, 32)])
        io = lax.broadcasted_iota(jnp.int32, (16,), 0)
        w0 = wout_v[pl.ds(0, 16)]
        w1 = wout_v[pl.ds(16, 16)]
        e_vec = jnp.zeros((16,), jnp.float32)
        for v in range(6):
            r0 = emb_v[pl.ds(v * 32, 16)]
            r1 = emb_v[pl.ds(v * 32 + 16, 16)]
            dot_v = jnp.sum(r0 * w0 + r1 * w1)
            e_vec = e_vec + jnp.where(io == v, dot_v, 0.0)
        e_vec = e_vec * (1.0 / _BP)
        elut_v[pl.ds(0, 16)] = plsc.bitcast(e_vec, jnp.int32)

        for j in range(_NIDX):
            copies[j].wait()

            def grp(g, carry, j=j):
                r0 = j * 128 + g * 16
                rows = r0 + io
                par = par_v[pl.ds(r0, 16)]
                acc = jnp.zeros((16,), jnp.float32)
                for bp in range(_BP):
                    col = par + ((io + bp) & (_BP - 1))
                    toks = plsc.load_gather(rows_v, [rows, col])
                    bits = plsc.load_gather(elut_v, [toks])
                    acc = acc + plsc.bitcast(bits, jnp.float32)
                s_v[pl.ds(r0, 16)] = acc
                return carry

            lax.fori_loop(0, 128 // 16, grp, 0)

        pltpu.sync_copy(s_v, s_hbm.at[pl.ds(base, _CHUNK)])

    return body


def _tc_combine(cu_ref, vals_ref, s_ref, out_ref):
    logv = jnp.log(vals_ref[...] + 1e-6)
    sv = s_ref[...]
    ls = logv * sv
    r_io = lax.broadcasted_iota(jnp.int32, (128, 128), 0)
    c_io = lax.broadcasted_iota(jnp.int32, (128, 128), 1)
    t_idx = r_io * 128 + c_io
    row16 = lax.broadcasted_iota(jnp.int32, (_NSEG, 128), 0)
    col16 = lax.broadcasted_iota(jnp.int32, (_NSEG, 128), 1)
    acc = jnp.zeros((_NSEG, 128), jnp.float32)
    for b in range(_NSEG):
        lo = cu_ref[b]
        hi = cu_ref[b + 1]
        m = ((t_idx >= lo) & (t_idx < hi)).astype(jnp.float32)
        cnt = jnp.sum(m)
        lsum = jnp.sum(m * logv)
        ssum = jnp.sum(m * sv)
        asum = jnp.sum(m * ls)
        res = asum - (lsum / jnp.maximum(cnt, 1.0)) * ssum
        acc = acc + jnp.where((row16 == b) & (col16 == 0), res, 0.0)
    out_ref[...] = acc


def kernel(values, cu_seqlens, feature_idx, token_table, embed_table, W_out):
    table2 = token_table.reshape(token_table.shape[0] // 2, 2 * _BP)
    s = _sc_scores_kernel()(
        feature_idx, table2,
        embed_table.reshape(-1), W_out.reshape(-1))
    out = pl.pallas_call(
        _tc_combine,
        out_shape=jax.ShapeDtypeStruct((_NSEG, 128), jnp.float32),
        in_specs=[
            pl.BlockSpec(memory_space=pltpu.SMEM),
            pl.BlockSpec(memory_space=pltpu.VMEM),
            pl.BlockSpec(memory_space=pltpu.VMEM),
        ],
    )(cu_seqlens, values.reshape(128, 128), s.reshape(128, 128))
    return out[:, :1]
